# baseline (speedup 1.0000x reference)
import functools

import jax
import jax.numpy as jnp
from jax import lax
from jax.experimental import pallas as pl
from jax.experimental.pallas import tpu as pltpu
from jax.experimental.pallas import tpu_sc as plsc

_NC = 2
_NS = 16
_LANES = 16
_PAD = 64
_MROWS = 8


def _sc_tail_sum(hidden, mask, xs):
    B, L, D = hidden.shape
    Dc = D // _NC
    nchunk = Dc // _LANES
    mesh = plsc.VectorSubcoreMesh(core_axis_name="c", subcore_axis_name="s")

    @functools.partial(
        pl.kernel,
        out_type=jax.ShapeDtypeStruct((B, 1, D), jnp.float32),
        mesh=mesh,
        compiler_params=pltpu.CompilerParams(needs_layout_passes=False),
        scratch_types=[
            pltpu.VMEM((_MROWS, L), jnp.int32),
            pltpu.VMEM((_LANES,), jnp.int32),
            pltpu.VMEM((_PAD, Dc), jnp.float32),
            pltpu.VMEM((Dc,), jnp.float32),
        ],
    )
    def k(hidden_hbm, mask_hbm, xs_hbm, out_hbm, mask_v, xs_v, win_v, acc_v):
        c = lax.axis_index("c")
        s = lax.axis_index("s")
        b = s
        b8 = (b // _MROWS) * _MROWS

        pltpu.sync_copy(mask_hbm.at[pl.ds(b8, _MROWS)], mask_v)
        pltpu.sync_copy(xs_hbm, xs_v)

        r = b - b8
        msum = jnp.zeros((_LANES,), jnp.int32)
        for i in range(L // _LANES):
            msum = msum + mask_v[r, pl.ds(i * _LANES, _LANES)]
        hi = jnp.sum(msum)
        x_s = jnp.max(xs_v[...])

        lo = jnp.maximum(hi - x_s, 0)
        base = jnp.minimum((lo // 8) * 8, L - _PAD)
        pltpu.sync_copy(
            hidden_hbm.at[b, pl.ds(base, _PAD), pl.ds(c * Dc, Dc)], win_v
        )

        lo_idx = lo - base
        hi_idx = hi - base

        def row(j, acc):
            return tuple(
                acc[t] + win_v[j, pl.ds(t * _LANES, _LANES)]
                for t in range(nchunk)
            )

        acc0 = tuple(jnp.zeros((_LANES,), jnp.float32) for _ in range(nchunk))
        acc = lax.fori_loop(lo_idx, hi_idx, row, acc0)
        for t in range(nchunk):
            acc_v[pl.ds(t * _LANES, _LANES)] = acc[t]
        pltpu.sync_copy(acc_v, out_hbm.at[b, 0, pl.ds(c * Dc, Dc)])

    return k(hidden, mask, xs)


def kernel(hidden, mask, x):
    B, L, D = hidden.shape
    assert B == _NS and D % (_NC * _LANES) == 0 and L % _LANES == 0
    assert L >= _PAD and L % 8 == 0 and B % _MROWS == 0
    xs = jnp.full((_LANES,), x, dtype=jnp.int32)
    out = _sc_tail_sum(hidden, mask.astype(jnp.int32), xs)
    return out.astype(hidden.dtype)


# speedup vs baseline: 1.0643x; 1.0643x over previous
import functools

import jax
import jax.numpy as jnp
from jax import lax
from jax.experimental import pallas as pl
from jax.experimental.pallas import tpu as pltpu
from jax.experimental.pallas import tpu_sc as plsc

_NC = 2
_NS = 16
_LANES = 16
_PAD = 64


def _sc_tail_sum(hidden, mask, xs):
    B, L, D = hidden.shape
    Dc = D // _NC
    nchunk = Dc // _LANES
    mesh = plsc.VectorSubcoreMesh(core_axis_name="c", subcore_axis_name="s")

    @functools.partial(
        pl.kernel,
        out_type=jax.ShapeDtypeStruct((B, 1, D), jnp.float32),
        mesh=mesh,
        compiler_params=pltpu.CompilerParams(
            needs_layout_passes=False,
            skip_device_barrier=True,
        ),
        scratch_types=[
            pltpu.VMEM((_LANES,), jnp.int32),
            pltpu.VMEM((1, L), jnp.int32),
            pltpu.VMEM((_LANES,), jnp.int32),
            pltpu.VMEM((_PAD, Dc), jnp.float32),
            pltpu.VMEM((Dc,), jnp.float32),
            pltpu.SemaphoreType.DMA,
        ],
    )
    def k(hidden_hbm, mask_hbm, xs_hbm, out_hbm,
          idx1_v, mask_v, xs_v, win_v, acc_v, sem):
        c = lax.axis_index("c")
        s = lax.axis_index("s")
        b = s

        bvec = jnp.zeros((_LANES,), jnp.int32) + b
        idx1_v[...] = bvec
        pltpu.async_copy(mask_hbm.at[idx1_v.at[pl.ds(0, 1)]], mask_v, sem).wait()
        pltpu.sync_copy(xs_hbm, xs_v)

        msum = jnp.zeros((_LANES,), jnp.int32)
        for i in range(L // _LANES):
            msum = msum + mask_v[0, pl.ds(i * _LANES, _LANES)]
        hi = jnp.sum(msum)
        x_s = jnp.max(xs_v[...])

        lo = jnp.maximum(hi - x_s, 0)
        base = jnp.minimum((lo // 8) * 8, L - _PAD)
        pltpu.sync_copy(
            hidden_hbm.at[b, pl.ds(base, _PAD), pl.ds(c * Dc, Dc)], win_v
        )

        lo_idx = lo - base
        hi_idx = hi - base

        def row(j, acc):
            return tuple(
                acc[t] + win_v[j, pl.ds(t * _LANES, _LANES)]
                for t in range(nchunk)
            )

        acc0 = tuple(jnp.zeros((_LANES,), jnp.float32) for _ in range(nchunk))
        acc = lax.fori_loop(lo_idx, hi_idx, row, acc0)
        for t in range(nchunk):
            acc_v[pl.ds(t * _LANES, _LANES)] = acc[t]
        pltpu.sync_copy(acc_v, out_hbm.at[b, 0, pl.ds(c * Dc, Dc)])

    return k(hidden, mask, xs)


def kernel(hidden, mask, x):
    B, L, D = hidden.shape
    assert B == _NS and D % (_NC * _LANES) == 0 and L % _LANES == 0
    assert L >= _PAD and L % 8 == 0
    xs = jnp.full((_LANES,), x, dtype=jnp.int32)
    out = _sc_tail_sum(hidden, mask.astype(jnp.int32), xs)
    return out.astype(hidden.dtype)


# speedup vs baseline: 1.0802x; 1.0150x over previous
import functools

import jax
import jax.numpy as jnp
from jax import lax
from jax.experimental import pallas as pl
from jax.experimental.pallas import tpu as pltpu
from jax.experimental.pallas import tpu_sc as plsc

_NC = 2
_NS = 16
_LANES = 16
_PAD = 64
_HALF = _PAD // 2


def _sc_tail_sum(hidden, mask, xs):
    B, L, D = hidden.shape
    Dc = D // _NC
    nchunk = Dc // _LANES
    mesh = plsc.VectorSubcoreMesh(core_axis_name="c", subcore_axis_name="s")

    @functools.partial(
        pl.kernel,
        out_type=jax.ShapeDtypeStruct((B, 1, D), jnp.float32),
        mesh=mesh,
        compiler_params=pltpu.CompilerParams(
            needs_layout_passes=False,
            skip_device_barrier=True,
        ),
        scratch_types=[
            pltpu.VMEM((_LANES,), jnp.int32),
            pltpu.VMEM((1, L), jnp.int32),
            pltpu.VMEM((_LANES,), jnp.int32),
            pltpu.VMEM((_PAD, Dc), jnp.float32),
            pltpu.VMEM((Dc,), jnp.float32),
            pltpu.SemaphoreType.DMA,
            pltpu.SemaphoreType.DMA,
            pltpu.SemaphoreType.DMA,
        ],
    )
    def k(hidden_hbm, mask_hbm, xs_hbm, out_hbm,
          idx1_v, mask_v, xs_v, win_v, acc_v, sem0, sem1, semx):
        c = lax.axis_index("c")
        s = lax.axis_index("s")
        b = s

        idx1_v[...] = jnp.zeros((_LANES,), jnp.int32) + b
        mcopy = pltpu.async_copy(
            mask_hbm.at[idx1_v.at[pl.ds(0, 1)]], mask_v, sem0
        )
        xcopy = pltpu.async_copy(xs_hbm, xs_v, semx)
        mcopy.wait()

        msum = jnp.zeros((_LANES,), jnp.int32)
        for i in range(L // _LANES):
            msum = msum + mask_v[0, pl.ds(i * _LANES, _LANES)]
        hi = jnp.sum(msum)
        xcopy.wait()
        x_s = jnp.max(xs_v[...])

        lo = jnp.maximum(hi - x_s, 0)
        base = jnp.minimum((lo // 8) * 8, L - _PAD)

        dcol = c * Dc
        cp0 = pltpu.async_copy(
            hidden_hbm.at[b, pl.ds(base, _HALF), pl.ds(dcol, Dc)],
            win_v.at[pl.ds(0, _HALF)], sem0,
        )
        cp1 = pltpu.async_copy(
            hidden_hbm.at[b, pl.ds(base + _HALF, _HALF), pl.ds(dcol, Dc)],
            win_v.at[pl.ds(_HALF, _HALF)], sem1,
        )

        lo_idx = lo - base
        hi_idx = hi - base

        def row(j, acc):
            return tuple(
                acc[t] + win_v[j, pl.ds(t * _LANES, _LANES)]
                for t in range(nchunk)
            )

        acc0 = tuple(jnp.zeros((_LANES,), jnp.float32) for _ in range(nchunk))
        cp0.wait()
        acc = lax.fori_loop(lo_idx, jnp.minimum(hi_idx, _HALF), row, acc0)
        cp1.wait()
        acc = lax.fori_loop(jnp.maximum(lo_idx, _HALF), hi_idx, row, acc)
        for t in range(nchunk):
            acc_v[pl.ds(t * _LANES, _LANES)] = acc[t]
        pltpu.sync_copy(acc_v, out_hbm.at[b, 0, pl.ds(dcol, Dc)])

    return k(hidden, mask, xs)


def kernel(hidden, mask, x):
    B, L, D = hidden.shape
    assert B == _NS and D % (_NC * _LANES) == 0 and L % _LANES == 0
    assert L >= _PAD and L % 8 == 0
    xs = jnp.full((_LANES,), x, dtype=jnp.int32)
    out = _sc_tail_sum(hidden, mask.astype(jnp.int32), xs)
    return out.astype(hidden.dtype)


# speedup vs baseline: 1.1416x; 1.0569x over previous
import functools

import jax
import jax.numpy as jnp
from jax import lax
from jax.experimental import pallas as pl
from jax.experimental.pallas import tpu as pltpu
from jax.experimental.pallas import tpu_sc as plsc

_NC = 2
_NS = 16
_LANES = 16
_PAD = 64


def _sc_tail_sum(hidden, mask, xs):
    B, L, D = hidden.shape
    Dc = D // _NC
    nchunk = Dc // _LANES
    mesh = plsc.VectorSubcoreMesh(core_axis_name="c", subcore_axis_name="s")

    @functools.partial(
        pl.kernel,
        out_type=jax.ShapeDtypeStruct((B, 1, D), jnp.float32),
        mesh=mesh,
        compiler_params=pltpu.CompilerParams(
            needs_layout_passes=False,
            skip_device_barrier=True,
        ),
        scratch_types=[
            pltpu.VMEM((_LANES,), jnp.int32),
            pltpu.VMEM((1, L), jnp.int32),
            pltpu.VMEM((_LANES,), jnp.int32),
            pltpu.VMEM((_PAD, Dc), jnp.float32),
            pltpu.VMEM((Dc,), jnp.float32),
            pltpu.SemaphoreType.DMA,
            pltpu.SemaphoreType.DMA,
            pltpu.SemaphoreType.DMA,
        ],
    )
    def k(hidden_hbm, mask_hbm, xs_hbm, out_hbm,
          idx1_v, mask_v, xs_v, win_v, acc_v, semw, semm, semx):
        c = lax.axis_index("c")
        s = lax.axis_index("s")
        b = s
        dcol = c * Dc

        wcopy = pltpu.async_copy(
            hidden_hbm.at[b, pl.ds(L - _PAD, _PAD), pl.ds(dcol, Dc)],
            win_v, semw,
        )

        idx1_v[...] = jnp.zeros((_LANES,), jnp.int32) + b
        mcopy = pltpu.async_copy(
            mask_hbm.at[idx1_v.at[pl.ds(0, 1)]], mask_v, semm
        )
        xcopy = pltpu.async_copy(xs_hbm, xs_v, semx)
        mcopy.wait()

        accs = [jnp.zeros((_LANES,), jnp.int32) for _ in range(4)]
        for i in range(L // (_LANES * 4)):
            for u in range(4):
                accs[u] = accs[u] + mask_v[0, pl.ds((4 * i + u) * _LANES, _LANES)]
        hi = jnp.sum(accs[0] + accs[1] + (accs[2] + accs[3]))
        xcopy.wait()
        x_s = jnp.max(xs_v[...])

        lo = jnp.maximum(hi - x_s, 0)
        base = jnp.minimum((lo // 8) * 8, L - _PAD)

        wcopy.wait()

        @pl.when(base != L - _PAD)
        def _respin():
            pltpu.sync_copy(
                hidden_hbm.at[b, pl.ds(base, _PAD), pl.ds(dcol, Dc)], win_v
            )

        lo_idx = lo - base
        hi_idx = hi - base

        def row(j, acc):
            return tuple(
                acc[t] + win_v[j, pl.ds(t * _LANES, _LANES)]
                for t in range(nchunk)
            )

        acc0 = tuple(jnp.zeros((_LANES,), jnp.float32) for _ in range(nchunk))
        acc = lax.fori_loop(lo_idx, hi_idx, row, acc0)
        for t in range(nchunk):
            acc_v[pl.ds(t * _LANES, _LANES)] = acc[t]
        pltpu.sync_copy(acc_v, out_hbm.at[b, 0, pl.ds(dcol, Dc)])

    return k(hidden, mask, xs)


def kernel(hidden, mask, x):
    B, L, D = hidden.shape
    assert B == _NS and D % (_NC * _LANES) == 0 and L % (_LANES * 4) == 0
    assert L >= _PAD and L % 8 == 0
    xs = jnp.full((_LANES,), x, dtype=jnp.int32)
    out = _sc_tail_sum(hidden, mask.astype(jnp.int32), xs)
    return out.astype(hidden.dtype)


# speedup vs baseline: 1.1469x; 1.0046x over previous
import functools

import jax
import jax.numpy as jnp
from jax import lax
from jax.experimental import pallas as pl
from jax.experimental.pallas import tpu as pltpu
from jax.experimental.pallas import tpu_sc as plsc

_NC = 2
_NS = 16
_LANES = 16
_PAD = 64


def _sc_tail_sum(hidden, mask, xs):
    B, L, D = hidden.shape
    Dc = D // _NC
    nchunk = Dc // _LANES
    mesh = plsc.VectorSubcoreMesh(core_axis_name="c", subcore_axis_name="s")

    @functools.partial(
        pl.kernel,
        out_type=jax.ShapeDtypeStruct((B, 1, D), jnp.float32),
        mesh=mesh,
        compiler_params=pltpu.CompilerParams(
            needs_layout_passes=False,
            skip_device_barrier=True,
        ),
        scratch_types=[
            pltpu.VMEM((_LANES,), jnp.int32),
            pltpu.VMEM((1, L), jnp.int32),
            pltpu.VMEM((_LANES,), jnp.int32),
            pltpu.VMEM((_PAD, Dc), jnp.float32),
            pltpu.VMEM((Dc,), jnp.float32),
            pltpu.SemaphoreType.DMA,
            pltpu.SemaphoreType.DMA,
            pltpu.SemaphoreType.DMA,
        ],
    )
    def k(hidden_hbm, mask_hbm, xs_hbm, out_hbm,
          idx1_v, mask_v, xs_v, win_v, acc_v, semw, semm, semx):
        c = lax.axis_index("c")
        s = lax.axis_index("s")
        b = s
        dcol = c * Dc

        wcopy = pltpu.async_copy(
            hidden_hbm.at[b, pl.ds(L - _PAD, _PAD), pl.ds(dcol, Dc)],
            win_v, semw,
        )

        idx1_v[...] = jnp.zeros((_LANES,), jnp.int32) + b
        mcopy = pltpu.async_copy(
            mask_hbm.at[idx1_v.at[pl.ds(0, 1)]], mask_v, semm
        )
        xcopy = pltpu.async_copy(xs_hbm, xs_v, semx)
        mcopy.wait()

        accs = [jnp.zeros((_LANES,), jnp.int32) for _ in range(4)]
        for i in range(L // (_LANES * 4)):
            for u in range(4):
                accs[u] = accs[u] + mask_v[0, pl.ds((4 * i + u) * _LANES, _LANES)]
        hi = jnp.sum(accs[0] + accs[1] + (accs[2] + accs[3]))
        xcopy.wait()
        x_s = jnp.max(xs_v[...])

        lo = jnp.maximum(hi - x_s, 0)
        base = jnp.minimum((lo // 8) * 8, L - _PAD)

        wcopy.wait()

        @pl.when(base != L - _PAD)
        def _respin():
            pltpu.sync_copy(
                hidden_hbm.at[b, pl.ds(base, _PAD), pl.ds(dcol, Dc)], win_v
            )

        lo_idx = lo - base
        hi_idx = hi - base
        nrows = hi_idx - lo_idx
        npairs = nrows // 2

        def row2(jj, acc):
            j0 = lo_idx + 2 * jj
            return tuple(
                acc[t]
                + win_v[j0, pl.ds(t * _LANES, _LANES)]
                + win_v[j0 + 1, pl.ds(t * _LANES, _LANES)]
                for t in range(nchunk)
            )

        acc0 = tuple(jnp.zeros((_LANES,), jnp.float32) for _ in range(nchunk))
        acc = lax.fori_loop(0, npairs, row2, acc0)
        jlast = jnp.minimum(lo_idx + 2 * npairs, _PAD - 1)
        wodd = (nrows - 2 * npairs).astype(jnp.float32)
        acc = tuple(
            acc[t] + wodd * win_v[jlast, pl.ds(t * _LANES, _LANES)]
            for t in range(nchunk)
        )
        for t in range(nchunk):
            acc_v[pl.ds(t * _LANES, _LANES)] = acc[t]
        pltpu.sync_copy(acc_v, out_hbm.at[b, 0, pl.ds(dcol, Dc)])

    return k(hidden, mask, xs)


def kernel(hidden, mask, x):
    B, L, D = hidden.shape
    assert B == _NS and D % (_NC * _LANES) == 0 and L % (_LANES * 4) == 0
    assert L >= _PAD and L % 8 == 0
    xs = jnp.full((_LANES,), x, dtype=jnp.int32)
    out = _sc_tail_sum(hidden, mask.astype(jnp.int32), xs)
    return out.astype(hidden.dtype)
